# baseline (speedup 1.0000x reference)
import jax
import jax.numpy as jnp
from jax.experimental import pallas as pl
from jax.experimental.pallas import tpu as pltpu

_N = 4096
_R = 2048
_K = 8


def _chamfer_body(ab_ref, fwd_ref, bwd_ref, fcol_ref):
    i = pl.program_id(0)
    d = jax.lax.dot_general(ab_ref[0:_K, pl.ds(i * _R, _R)], ab_ref[_K:, :],
                            (((0,), (0,)), ((), ())),
                            preferred_element_type=jnp.float32)
    fcol_ref[pl.ds(i * _R, _R), :] = jnp.maximum(
        jnp.min(d, axis=1, keepdims=True), 0.0)
    colmin = jnp.min(d, axis=0)
    last = _N // _R - 1

    @pl.when(i == last)
    def _():
        fwd_ref[...] = fcol_ref[...].reshape(1, _N)[0]

    @pl.when(i == 0)
    def _():
        bwd_ref[...] = colmin

    @pl.when((i > 0) & (i < last))
    def _():
        bwd_ref[...] = jnp.minimum(bwd_ref[...], colmin)

    @pl.when(i == last)
    def _():
        bwd_ref[...] = jnp.maximum(jnp.minimum(bwd_ref[...], colmin), 0.0)


def _bf16_hi_lo(x):
    hi = jax.lax.bitcast_convert_type(
        jax.lax.bitcast_convert_type(x, jnp.uint32) & jnp.uint32(0xFFFF0000),
        jnp.float32)
    return hi, x - hi


def _augment_t(pts, is_source):
    sq = jnp.sum(pts * pts, axis=1)
    sq_hi, sq_lo = _bf16_hi_lo(sq)
    ones = jnp.ones_like(sq)
    zero = jnp.zeros_like(sq)
    x, y, z = pts[:, 0], pts[:, 1], pts[:, 2]
    if is_source:
        rows = [sq_hi, sq_lo, ones, ones, -2.0 * x, -2.0 * y, -2.0 * z, zero]
    else:
        rows = [ones, ones, sq_hi, sq_lo, x, y, z, zero]
    return jnp.stack(rows, axis=0).astype(jnp.bfloat16)


def kernel(source_cloud, target_cloud):
    a_t = _augment_t(source_cloud[0], True)
    b_t = _augment_t(target_cloud[0], False)

    ab = jnp.concatenate([a_t, b_t], axis=0)

    fwd, bwd = pl.pallas_call(
        _chamfer_body,
        grid=(_N // _R,),
        in_specs=[
            pl.BlockSpec((2 * _K, _N), lambda i: (0, 0)),
        ],
        out_specs=[
            pl.BlockSpec((_N,), lambda i: (0,)),
            pl.BlockSpec((_N,), lambda i: (0,)),
        ],
        out_shape=[
            jax.ShapeDtypeStruct((_N,), jnp.float32),
            jax.ShapeDtypeStruct((_N,), jnp.float32),
        ],
        scratch_shapes=[
            pltpu.VMEM((_N, 1), jnp.float32),
        ],
    )(ab)

    return fwd, bwd


# speedup vs baseline: 1.0027x; 1.0027x over previous
import jax
import jax.numpy as jnp
from jax.experimental import pallas as pl
from jax.experimental.pallas import tpu as pltpu

_N = 4096
_R = 2048
_K = 8


def _chamfer_body(ab_ref, fwd_ref, bwd_ref, fcol_ref):
    i = pl.program_id(0)
    d = jax.lax.dot_general(ab_ref[0:_K, pl.ds(i * _R, _R)], ab_ref[_K:, :],
                            (((0,), (0,)), ((), ())),
                            preferred_element_type=jnp.float32)
    fcol_ref[pl.ds(i * _R, _R), :] = jnp.maximum(
        jnp.min(d, axis=1, keepdims=True), 0.0)
    colmin = jnp.min(d, axis=0)
    last = _N // _R - 1

    @pl.when(i == last)
    def _():
        fwd_ref[...] = fcol_ref[...].reshape(1, _N)[0]

    @pl.when(i == 0)
    def _():
        bwd_ref[...] = colmin

    @pl.when((i > 0) & (i < last))
    def _():
        bwd_ref[...] = jnp.minimum(bwd_ref[...], colmin)

    @pl.when(i == last)
    def _():
        bwd_ref[...] = jnp.maximum(jnp.minimum(bwd_ref[...], colmin), 0.0)


def _bf16_hi_lo(x):
    hi = jax.lax.bitcast_convert_type(
        jax.lax.bitcast_convert_type(x, jnp.uint32) & jnp.uint32(0xFFFF0000),
        jnp.float32)
    return hi, x - hi


def _factor_rows(pts, is_source):
    sq = jnp.sum(pts * pts, axis=1)
    sq_hi, sq_lo = _bf16_hi_lo(sq)
    ones = jnp.ones_like(sq)
    zero = jnp.zeros_like(sq)
    x, y, z = pts[:, 0], pts[:, 1], pts[:, 2]
    if is_source:
        return [sq_hi, sq_lo, ones, ones, -2.0 * x, -2.0 * y, -2.0 * z, zero]
    return [ones, ones, sq_hi, sq_lo, x, y, z, zero]


def kernel(source_cloud, target_cloud):
    rows = (_factor_rows(source_cloud[0], True)
            + _factor_rows(target_cloud[0], False))
    ab = jnp.stack(rows, axis=0).astype(jnp.bfloat16)

    fwd, bwd = pl.pallas_call(
        _chamfer_body,
        grid=(_N // _R,),
        in_specs=[
            pl.BlockSpec((2 * _K, _N), lambda i: (0, 0)),
        ],
        out_specs=[
            pl.BlockSpec((_N,), lambda i: (0,)),
            pl.BlockSpec((_N,), lambda i: (0,)),
        ],
        out_shape=[
            jax.ShapeDtypeStruct((_N,), jnp.float32),
            jax.ShapeDtypeStruct((_N,), jnp.float32),
        ],
        scratch_shapes=[
            pltpu.VMEM((_N, 1), jnp.float32),
        ],
    )(ab)

    return fwd, bwd
